# baseline (speedup 1.0000x reference)
import jax
import jax.numpy as jnp
from jax import lax
from jax.experimental import pallas as pl
from jax.experimental.pallas import tpu as pltpu
from jax.experimental.pallas import tpu_sc as plsc

N = 32768
NS = 16
LANES = 16
CHUNK = N // NS
HALF = CHUNK // 2
NV = CHUNK // LANES
NACC = 8
BIG = 2**30

_mesh = plsc.VectorSubcoreMesh(
    core_axis_name="c", subcore_axis_name="s", num_cores=1)


def _merge_pairs(a, b):
    av, ai = a
    bv, bi = b
    take_b = (bv > av) | ((bv == av) & (bi < ai))
    return jnp.where(take_b, bv, av), jnp.where(take_b, bi, ai)


def _tree_min(vs):
    while len(vs) > 1:
        vs = [jnp.minimum(vs[i], vs[i + 1]) for i in range(0, len(vs) - 1, 2)] \
            + ([vs[-1]] if len(vs) % 2 else [])
    return vs[0]


def _tree_max(vs):
    while len(vs) > 1:
        vs = [jnp.maximum(vs[i], vs[i + 1]) for i in range(0, len(vs) - 1, 2)] \
            + ([vs[-1]] if len(vs) % 2 else [])
    return vs[0]


@pl.kernel(
    mesh=_mesh,
    out_type=jax.ShapeDtypeStruct((LANES,), jnp.float32),
    compiler_params=pltpu.CompilerParams(
        needs_layout_passes=False, use_tc_tiling_on_sc=False),
    scratch_types=[
        pltpu.VMEM((CHUNK,), jnp.float32),
        pltpu.VMEM((2, LANES), jnp.int32),
        pltpu.VMEM((NS, 2, LANES), jnp.int32),
        pltpu.VMEM_SHARED((NS, 2, LANES), jnp.int32),
        pltpu.VMEM((LANES,), jnp.float32),
        pltpu.SemaphoreType.DMA,
        pltpu.SemaphoreType.DMA,
    ],
)
def _top1_match_kernel(x_hbm, out_hbm, chunk, stage, lall, shared, outv,
                       sem0, sem1):
    cid = lax.axis_index("c")
    sid = lax.axis_index("s")
    base = sid * CHUNK

    cp0 = pltpu.async_copy(
        x_hbm.at[pl.ds(base, HALF)], chunk.at[pl.ds(0, HALF)], sem0)
    cp1 = pltpu.async_copy(
        x_hbm.at[pl.ds(base + HALF, HALF)], chunk.at[pl.ds(HALF, HALF)], sem1)

    ninf = jnp.full((LANES,), -jnp.inf, jnp.float32)
    zero = jnp.zeros((LANES,), jnp.int32)
    init = tuple((ninf, zero) for _ in range(NACC))

    def sweep(lo, hi, carry_in):
        @plsc.parallel_loop(lo, hi, step=NACC, unroll=2, carry=carry_in)
        def accs(j, carry):
            out = []
            for k in range(NACC):
                vmax, vidx = carry[k]
                v = chunk[pl.ds((j + k) * LANES, LANES)]
                take = v > vmax
                out.append((jnp.where(take, v, vmax),
                            jnp.where(take,
                                      jnp.full((LANES,), j + k, jnp.int32),
                                      vidx)))
            return tuple(out)
        return accs

    cp0.wait()
    mid = sweep(0, NV // 2, init)
    cp1.wait()
    accs = sweep(NV // 2, NV, mid)

    pairs = list(accs)
    while len(pairs) > 1:
        pairs = [_merge_pairs(pairs[i], pairs[i + 1])
                 for i in range(0, len(pairs), 2)]
    vmax, vidx = pairs[0]
    pubidx = vidx * LANES + (lax.iota(jnp.int32, LANES) + base)

    stage[0, :] = plsc.bitcast(vmax, jnp.int32)
    stage[1, :] = pubidx
    pltpu.sync_copy(stage, shared.at[sid])
    plsc.subcore_barrier()

    @pl.when((cid == 0) & (sid == 0))
    def _():
        pltpu.sync_copy(shared, lall)
        rows_max = [plsc.bitcast(lall[i, 0], jnp.float32) for i in range(NS)]
        rows_idx = [lall[i, 1] for i in range(NS)]
        gmax = jnp.max(_tree_max(rows_max))
        gsplat = jnp.broadcast_to(gmax, (LANES,))
        bigv = jnp.full((LANES,), BIG, jnp.int32)

        cand_a = [jnp.where(rows_max[i] == gsplat, rows_idx[i], bigv)
                  for i in range(NS)]
        top1 = jnp.min(_tree_min(cand_a))
        cand_b = [jnp.where(rows_max[i] == gsplat, rows_idx[i], bigv)
                  for i in range(NS)]
        first_occ = jnp.min(_tree_min(cand_b))

        result = jnp.where(top1 == first_occ, 1.0, 0.0).astype(jnp.float32)
        outv[...] = jnp.broadcast_to(result, (LANES,))
        pltpu.sync_copy(outv, out_hbm)


def kernel(x):
    return _top1_match_kernel(x)[:1]


# speedup vs baseline: 1.0028x; 1.0028x over previous
import jax
import jax.numpy as jnp
from jax import lax
from jax.experimental import pallas as pl
from jax.experimental.pallas import tpu as pltpu
from jax.experimental.pallas import tpu_sc as plsc

N = 32768
NS = 16
LANES = 16
CHUNK = N // NS
HALF = CHUNK // 2
NV = CHUNK // LANES
NACC = 4
BIG = 2**30

_mesh = plsc.VectorSubcoreMesh(
    core_axis_name="c", subcore_axis_name="s", num_cores=1)


def _merge_pairs(a, b):
    av, ai = a
    bv, bi = b
    take_b = (bv > av) | ((bv == av) & (bi < ai))
    return jnp.where(take_b, bv, av), jnp.where(take_b, bi, ai)


def _tree_min(vs):
    while len(vs) > 1:
        vs = [jnp.minimum(vs[i], vs[i + 1]) for i in range(0, len(vs) - 1, 2)] \
            + ([vs[-1]] if len(vs) % 2 else [])
    return vs[0]


def _tree_max(vs):
    while len(vs) > 1:
        vs = [jnp.maximum(vs[i], vs[i + 1]) for i in range(0, len(vs) - 1, 2)] \
            + ([vs[-1]] if len(vs) % 2 else [])
    return vs[0]


@pl.kernel(
    mesh=_mesh,
    out_type=jax.ShapeDtypeStruct((LANES,), jnp.float32),
    compiler_params=pltpu.CompilerParams(
        needs_layout_passes=False, use_tc_tiling_on_sc=False),
    scratch_types=[
        pltpu.VMEM((CHUNK,), jnp.float32),
        pltpu.VMEM((2, LANES), jnp.int32),
        pltpu.VMEM((NS, 2, LANES), jnp.int32),
        pltpu.VMEM_SHARED((NS, 2, LANES), jnp.int32),
        pltpu.VMEM((LANES,), jnp.float32),
        pltpu.SemaphoreType.DMA,
        pltpu.SemaphoreType.DMA,
    ],
)
def _top1_match_kernel(x_hbm, out_hbm, chunk, stage, lall, shared, outv,
                       sem0, sem1):
    cid = lax.axis_index("c")
    sid = lax.axis_index("s")
    base = sid * CHUNK

    cp0 = pltpu.async_copy(
        x_hbm.at[pl.ds(base, HALF)], chunk.at[pl.ds(0, HALF)], sem0)
    cp1 = pltpu.async_copy(
        x_hbm.at[pl.ds(base + HALF, HALF)], chunk.at[pl.ds(HALF, HALF)], sem1)

    ninf = jnp.full((LANES,), -jnp.inf, jnp.float32)
    zero = jnp.zeros((LANES,), jnp.int32)
    init = tuple((ninf, zero) for _ in range(NACC))

    def sweep(lo, hi, carry_in):
        @plsc.parallel_loop(lo, hi, step=NACC, unroll=2, carry=carry_in)
        def accs(j, carry):
            out = []
            for k in range(NACC):
                vmax, vidx = carry[k]
                v = chunk[pl.ds((j + k) * LANES, LANES)]
                take = v > vmax
                out.append((jnp.where(take, v, vmax),
                            jnp.where(take,
                                      jnp.full((LANES,), j + k, jnp.int32),
                                      vidx)))
            return tuple(out)
        return accs

    cp0.wait()
    mid = sweep(0, NV // 2, init)
    cp1.wait()
    accs = sweep(NV // 2, NV, mid)

    pairs = list(accs)
    while len(pairs) > 1:
        pairs = [_merge_pairs(pairs[i], pairs[i + 1])
                 for i in range(0, len(pairs), 2)]
    vmax, vidx = pairs[0]
    pubidx = vidx * LANES + (lax.iota(jnp.int32, LANES) + base)

    stage[0, :] = plsc.bitcast(vmax, jnp.int32)
    stage[1, :] = pubidx
    pltpu.sync_copy(stage, shared.at[sid])
    plsc.subcore_barrier()

    @pl.when((cid == 0) & (sid == 0))
    def _():
        pltpu.sync_copy(shared, lall)
        rows_max = [plsc.bitcast(lall[i, 0], jnp.float32) for i in range(NS)]
        rows_idx = [lall[i, 1] for i in range(NS)]
        gmax = jnp.max(_tree_max(rows_max))
        gsplat = jnp.broadcast_to(gmax, (LANES,))
        bigv = jnp.full((LANES,), BIG, jnp.int32)

        cand_a = [jnp.where(rows_max[i] == gsplat, rows_idx[i], bigv)
                  for i in range(NS)]
        top1 = jnp.min(_tree_min(cand_a))
        cand_b = [jnp.where(rows_max[i] == gsplat, rows_idx[i], bigv)
                  for i in range(NS)]
        first_occ = jnp.min(_tree_min(cand_b))

        result = jnp.where(top1 == first_occ, 1.0, 0.0).astype(jnp.float32)
        outv[...] = jnp.broadcast_to(result, (LANES,))
        pltpu.sync_copy(outv, out_hbm)


def kernel(x):
    return _top1_match_kernel(x)[:1]
